# Initial kernel scaffold; baseline (speedup 1.0000x reference)
import jax
import jax.numpy as jnp
from jax.experimental import pallas as pl


def kernel(x, W, gamma, beta):
    raise NotImplementedError("write your pallas kernel here")


# speedup vs baseline: 6.8155x; 6.8155x over previous
import functools

import jax
import jax.numpy as jnp
from jax.experimental import pallas as pl

_K = 20


def _edge_kernel(x_ref, xt_ref, xtb_ref, wt_ref,
                 amax_ref, amin_ref, s1p_ref, s2p_ref):
    xb = x_ref[0]
    xf = xt_ref[0]
    xn = xtb_ref[0]
    C, N = xb.shape
    BR = xn.shape[0]

    xxf = jnp.sum(xb * xb, axis=0, keepdims=True)
    xxn = jnp.sum(xn * xn, axis=1, keepdims=True)
    d = 2.0 * jax.lax.dot_general(
        xn, xb, (((1,), (0,)), ((), ())), preferred_element_type=jnp.float32)
    d = d - xxf - xxn

    w1 = wt_ref[0:C, :]
    w2 = wt_ref[C:2 * C, :]
    y1t = jax.lax.dot_general(
        xf, w1, (((1,), (0,)), ((), ())), preferred_element_type=jnp.float32)
    y2t = jax.lax.dot_general(
        xn, w2 - w1, (((1,), (0,)), ((), ())),
        preferred_element_type=jnp.float32)

    iota = jax.lax.broadcasted_iota(jnp.int32, (BR, N), 1)
    work = d
    s1 = jnp.zeros_like(y2t)
    s2 = jnp.zeros_like(y2t)
    vmax = jnp.full_like(y2t, -jnp.inf)
    vmin = jnp.full_like(y2t, jnp.inf)
    for _ in range(_K):
        m = jnp.max(work, axis=1, keepdims=True)
        eq = work == m
        jsel = jnp.min(jnp.where(eq, iota, N), axis=1, keepdims=True)
        onehot = iota == jsel
        work = jnp.where(onehot, -jnp.inf, work)
        g = jax.lax.dot_general(
            onehot.astype(jnp.float32), y1t, (((1,), (0,)), ((), ())),
            preferred_element_type=jnp.float32)
        s1 = s1 + g
        s2 = s2 + g * g
        vmax = jnp.maximum(vmax, g)
        vmin = jnp.minimum(vmin, g)

    amax_ref[0] = y2t + vmax
    amin_ref[0] = y2t + vmin
    kf = float(_K)
    s1p_ref[0, 0, 0] = jnp.sum(s1 + kf * y2t, axis=0)
    s2p_ref[0, 0, 0] = jnp.sum(s2 + 2.0 * y2t * s1 + kf * (y2t * y2t), axis=0)


def _finish_kernel(amax_ref, amin_ref, s1p_ref, s2p_ref, gamma_ref, beta_ref,
                   out_ref, *, total):
    s1 = jnp.sum(s1p_ref[...], axis=(0, 1, 2))
    s2 = jnp.sum(s2p_ref[...], axis=(0, 1, 2))
    mean = s1 / total
    var = s2 / total - mean * mean
    inv = jax.lax.rsqrt(var + 1e-5)
    scale = gamma_ref[0] * inv
    shift = beta_ref[0] - mean * scale
    a = jnp.where(scale[None, :] >= 0.0, amax_ref[0], amin_ref[0])
    v = a * scale[None, :] + shift[None, :]
    v = jnp.where(v > 0.0, v, 0.2 * v)
    out_ref[0] = v.T


def kernel(x, W, gamma, beta):
    B, C, N = x.shape
    O = W.shape[0]
    BR = min(256, N)
    T = N // BR
    xt = jnp.transpose(x, (0, 2, 1))
    wt = jnp.transpose(W)
    grid = (B, T)

    amax, amin, s1p, s2p = pl.pallas_call(
        _edge_kernel,
        grid=grid,
        in_specs=[
            pl.BlockSpec((1, C, N), lambda b, t: (b, 0, 0)),
            pl.BlockSpec((1, N, C), lambda b, t: (b, 0, 0)),
            pl.BlockSpec((1, BR, C), lambda b, t: (b, t, 0)),
            pl.BlockSpec((2 * C, O), lambda b, t: (0, 0)),
        ],
        out_specs=[
            pl.BlockSpec((1, BR, O), lambda b, t: (b, t, 0)),
            pl.BlockSpec((1, BR, O), lambda b, t: (b, t, 0)),
            pl.BlockSpec((1, 1, 1, O), lambda b, t: (b, t, 0, 0)),
            pl.BlockSpec((1, 1, 1, O), lambda b, t: (b, t, 0, 0)),
        ],
        out_shape=[
            jax.ShapeDtypeStruct((B, N, O), jnp.float32),
            jax.ShapeDtypeStruct((B, N, O), jnp.float32),
            jax.ShapeDtypeStruct((B, T, 1, O), jnp.float32),
            jax.ShapeDtypeStruct((B, T, 1, O), jnp.float32),
        ],
    )(x, xt, xt, wt)

    total = float(B * N * _K)
    out = pl.pallas_call(
        functools.partial(_finish_kernel, total=total),
        grid=grid,
        in_specs=[
            pl.BlockSpec((1, BR, O), lambda b, t: (b, t, 0)),
            pl.BlockSpec((1, BR, O), lambda b, t: (b, t, 0)),
            pl.BlockSpec((B, T, 1, O), lambda b, t: (0, 0, 0, 0)),
            pl.BlockSpec((B, T, 1, O), lambda b, t: (0, 0, 0, 0)),
            pl.BlockSpec((1, O), lambda b, t: (0, 0)),
            pl.BlockSpec((1, O), lambda b, t: (0, 0)),
        ],
        out_specs=pl.BlockSpec((1, O, BR), lambda b, t: (b, 0, t)),
        out_shape=jax.ShapeDtypeStruct((B, O, N), jnp.float32),
    )(amax, amin, s1p, s2p, gamma.reshape(1, -1), beta.reshape(1, -1))
    return out


# speedup vs baseline: 9.0620x; 1.3296x over previous
import functools

import jax
import jax.numpy as jnp
from jax.experimental import pallas as pl

_K = 20


def _edge_kernel(x_ref, xt_ref, xtb_ref, wt_ref,
                 amax_ref, amin_ref, s1p_ref, s2p_ref):
    xb = x_ref[0]
    xf = xt_ref[0]
    xn = xtb_ref[0]
    C, N = xb.shape
    BR = xn.shape[0]

    xxf = jnp.sum(xb * xb, axis=0, keepdims=True)
    xxn = jnp.sum(xn * xn, axis=1, keepdims=True)
    d = 2.0 * jax.lax.dot_general(
        xn, xb, (((1,), (0,)), ((), ())), preferred_element_type=jnp.float32)
    d = d - xxf - xxn

    w1 = wt_ref[0:C, :]
    w2 = wt_ref[C:2 * C, :]
    y1t = jax.lax.dot_general(
        xf, w1, (((1,), (0,)), ((), ())), preferred_element_type=jnp.float32)
    y2t = jax.lax.dot_general(
        xn, w2 - w1, (((1,), (0,)), ((), ())),
        preferred_element_type=jnp.float32)

    iota = jax.lax.broadcasted_iota(jnp.int32, (BR, N), 1)
    work = d
    s1 = jnp.zeros_like(y2t)
    s2 = jnp.zeros_like(y2t)
    vmax = jnp.full_like(y2t, -jnp.inf)
    vmin = jnp.full_like(y2t, jnp.inf)
    for _ in range(_K):
        jsel = jnp.argmax(work, axis=1)
        onehot = iota == jsel[:, None]
        work = jnp.where(onehot, -jnp.inf, work)
        g = jax.lax.dot_general(
            onehot.astype(jnp.float32), y1t, (((1,), (0,)), ((), ())),
            preferred_element_type=jnp.float32)
        s1 = s1 + g
        s2 = s2 + g * g
        vmax = jnp.maximum(vmax, g)
        vmin = jnp.minimum(vmin, g)

    amax_ref[0] = y2t + vmax
    amin_ref[0] = y2t + vmin
    kf = float(_K)
    s1p_ref[0, 0, 0] = jnp.sum(s1 + kf * y2t, axis=0)
    s2p_ref[0, 0, 0] = jnp.sum(s2 + 2.0 * y2t * s1 + kf * (y2t * y2t), axis=0)


def _finish_kernel(amax_ref, amin_ref, s1p_ref, s2p_ref, gamma_ref, beta_ref,
                   out_ref, *, total):
    s1 = jnp.sum(s1p_ref[...], axis=(0, 1, 2))
    s2 = jnp.sum(s2p_ref[...], axis=(0, 1, 2))
    mean = s1 / total
    var = s2 / total - mean * mean
    inv = jax.lax.rsqrt(var + 1e-5)
    scale = gamma_ref[0] * inv
    shift = beta_ref[0] - mean * scale
    a = jnp.where(scale[None, :] >= 0.0, amax_ref[0], amin_ref[0])
    v = a * scale[None, :] + shift[None, :]
    v = jnp.where(v > 0.0, v, 0.2 * v)
    out_ref[0] = v.T


def kernel(x, W, gamma, beta):
    B, C, N = x.shape
    O = W.shape[0]
    BR = min(256, N)
    T = N // BR
    xt = jnp.transpose(x, (0, 2, 1))
    wt = jnp.transpose(W)
    grid = (B, T)

    amax, amin, s1p, s2p = pl.pallas_call(
        _edge_kernel,
        grid=grid,
        in_specs=[
            pl.BlockSpec((1, C, N), lambda b, t: (b, 0, 0)),
            pl.BlockSpec((1, N, C), lambda b, t: (b, 0, 0)),
            pl.BlockSpec((1, BR, C), lambda b, t: (b, t, 0)),
            pl.BlockSpec((2 * C, O), lambda b, t: (0, 0)),
        ],
        out_specs=[
            pl.BlockSpec((1, BR, O), lambda b, t: (b, t, 0)),
            pl.BlockSpec((1, BR, O), lambda b, t: (b, t, 0)),
            pl.BlockSpec((1, 1, 1, O), lambda b, t: (b, t, 0, 0)),
            pl.BlockSpec((1, 1, 1, O), lambda b, t: (b, t, 0, 0)),
        ],
        out_shape=[
            jax.ShapeDtypeStruct((B, N, O), jnp.float32),
            jax.ShapeDtypeStruct((B, N, O), jnp.float32),
            jax.ShapeDtypeStruct((B, T, 1, O), jnp.float32),
            jax.ShapeDtypeStruct((B, T, 1, O), jnp.float32),
        ],
    )(x, xt, xt, wt)

    total = float(B * N * _K)
    out = pl.pallas_call(
        functools.partial(_finish_kernel, total=total),
        grid=grid,
        in_specs=[
            pl.BlockSpec((1, BR, O), lambda b, t: (b, t, 0)),
            pl.BlockSpec((1, BR, O), lambda b, t: (b, t, 0)),
            pl.BlockSpec((B, T, 1, O), lambda b, t: (0, 0, 0, 0)),
            pl.BlockSpec((B, T, 1, O), lambda b, t: (0, 0, 0, 0)),
            pl.BlockSpec((1, O), lambda b, t: (0, 0)),
            pl.BlockSpec((1, O), lambda b, t: (0, 0)),
        ],
        out_specs=pl.BlockSpec((1, O, BR), lambda b, t: (b, 0, t)),
        out_shape=jax.ShapeDtypeStruct((B, O, N), jnp.float32),
    )(amax, amin, s1p, s2p, gamma.reshape(1, -1), beta.reshape(1, -1))
    return out


# speedup vs baseline: 9.3529x; 1.0321x over previous
import functools

import jax
import jax.numpy as jnp
from jax.experimental import pallas as pl

_K = 20
_TN = (((0,), (0,)), ((), ()))


def _edge_kernel(x_ref, xb_ref, wt_ref,
                 amax_ref, amin_ref, s1p_ref, s2p_ref):
    xb = x_ref[0]
    xc = xb_ref[0]
    C, N = xb.shape
    BR = xc.shape[1]

    xxf = jnp.sum(xb * xb, axis=0, keepdims=True)
    xxn = jnp.sum(xc * xc, axis=0)[:, None]
    d = 2.0 * jax.lax.dot_general(
        xc, xb, _TN, preferred_element_type=jnp.float32)
    d = d - xxf - xxn

    w1 = wt_ref[0:C, :]
    w2 = wt_ref[C:2 * C, :]
    y1t = jax.lax.dot_general(
        xb, w1, _TN, preferred_element_type=jnp.float32)
    y2t = jax.lax.dot_general(
        xc, w2 - w1, _TN, preferred_element_type=jnp.float32)

    iota = jax.lax.broadcasted_iota(jnp.int32, (BR, N), 1)
    work = d
    s1 = jnp.zeros_like(y2t)
    s2 = jnp.zeros_like(y2t)
    vmax = jnp.full_like(y2t, -jnp.inf)
    vmin = jnp.full_like(y2t, jnp.inf)
    for _ in range(_K):
        jsel = jnp.argmax(work, axis=1)
        onehot = iota == jsel[:, None]
        work = jnp.where(onehot, -jnp.inf, work)
        g = jax.lax.dot_general(
            onehot.astype(jnp.float32), y1t, (((1,), (0,)), ((), ())),
            preferred_element_type=jnp.float32)
        s1 = s1 + g
        s2 = s2 + g * g
        vmax = jnp.maximum(vmax, g)
        vmin = jnp.minimum(vmin, g)

    amax_ref[0] = y2t + vmax
    amin_ref[0] = y2t + vmin
    kf = float(_K)
    s1p_ref[0, 0, 0] = jnp.sum(s1 + kf * y2t, axis=0)
    s2p_ref[0, 0, 0] = jnp.sum(s2 + 2.0 * y2t * s1 + kf * (y2t * y2t), axis=0)


def _finish_kernel(amax_ref, amin_ref, s1p_ref, s2p_ref, gamma_ref, beta_ref,
                   out_ref, *, total):
    s1 = jnp.sum(s1p_ref[...], axis=(0, 1, 2))
    s2 = jnp.sum(s2p_ref[...], axis=(0, 1, 2))
    mean = s1 / total
    var = s2 / total - mean * mean
    inv = jax.lax.rsqrt(var + 1e-5)
    scale = gamma_ref[0] * inv
    shift = beta_ref[0] - mean * scale
    a = jnp.where(scale[None, :] >= 0.0, amax_ref[0], amin_ref[0])
    v = a * scale[None, :] + shift[None, :]
    v = jnp.where(v > 0.0, v, 0.2 * v)
    out_ref[0] = v.T


def kernel(x, W, gamma, beta):
    B, C, N = x.shape
    O = W.shape[0]
    BR = min(256, N)
    T = N // BR
    wt = jnp.transpose(W)

    amax, amin, s1p, s2p = pl.pallas_call(
        _edge_kernel,
        grid=(B, T),
        in_specs=[
            pl.BlockSpec((1, C, N), lambda b, t: (b, 0, 0)),
            pl.BlockSpec((1, C, BR), lambda b, t: (b, 0, t)),
            pl.BlockSpec((2 * C, O), lambda b, t: (0, 0)),
        ],
        out_specs=[
            pl.BlockSpec((1, BR, O), lambda b, t: (b, t, 0)),
            pl.BlockSpec((1, BR, O), lambda b, t: (b, t, 0)),
            pl.BlockSpec((1, 1, 1, O), lambda b, t: (b, t, 0, 0)),
            pl.BlockSpec((1, 1, 1, O), lambda b, t: (b, t, 0, 0)),
        ],
        out_shape=[
            jax.ShapeDtypeStruct((B, N, O), jnp.float32),
            jax.ShapeDtypeStruct((B, N, O), jnp.float32),
            jax.ShapeDtypeStruct((B, T, 1, O), jnp.float32),
            jax.ShapeDtypeStruct((B, T, 1, O), jnp.float32),
        ],
    )(x, x, wt)

    total = float(B * N * _K)
    out = pl.pallas_call(
        functools.partial(_finish_kernel, total=total),
        grid=(B,),
        in_specs=[
            pl.BlockSpec((1, N, O), lambda b: (b, 0, 0)),
            pl.BlockSpec((1, N, O), lambda b: (b, 0, 0)),
            pl.BlockSpec((B, T, 1, O), lambda b: (0, 0, 0, 0)),
            pl.BlockSpec((B, T, 1, O), lambda b: (0, 0, 0, 0)),
            pl.BlockSpec((1, O), lambda b: (0, 0)),
            pl.BlockSpec((1, O), lambda b: (0, 0)),
        ],
        out_specs=pl.BlockSpec((1, O, N), lambda b: (b, 0, 0)),
        out_shape=jax.ShapeDtypeStruct((B, O, N), jnp.float32),
    )(amax, amin, s1p, s2p, gamma.reshape(1, -1), beta.reshape(1, -1))
    return out


# speedup vs baseline: 9.7623x; 1.0438x over previous
import functools

import jax
import jax.numpy as jnp
from jax.experimental import pallas as pl

_K = 20
_TN = (((0,), (0,)), ((), ()))


def _edge_kernel(x_ref, xb_ref, wt_ref,
                 amax_ref, amin_ref, s1p_ref, s2p_ref):
    xb = x_ref[0]
    xc = xb_ref[0]
    C, N = xb.shape
    BR = xc.shape[1]

    xxf = jnp.sum(xb * xb, axis=0, keepdims=True)
    xxn = jnp.sum(xc * xc, axis=0)[:, None]
    d = 2.0 * jax.lax.dot_general(
        xc, xb, _TN, preferred_element_type=jnp.float32)
    d = d - xxf - xxn

    w1 = wt_ref[0:C, :]
    w2 = wt_ref[C:2 * C, :]
    y1t = jax.lax.dot_general(
        xb, w1, _TN, preferred_element_type=jnp.float32)
    y2t = jax.lax.dot_general(
        xc, w2 - w1, _TN, preferred_element_type=jnp.float32)

    g0 = jax.lax.dot_general(
        xc, w1, _TN, preferred_element_type=jnp.float32)
    iota = jax.lax.broadcasted_iota(jnp.int32, (BR, N), 1)
    riota = jax.lax.broadcasted_iota(jnp.int32, (BR, 1), 0)
    off = pl.program_id(1) * BR
    work = jnp.where(iota == riota + off, -jnp.inf, d)
    s1 = g0
    s2 = g0 * g0
    vmax = g0
    vmin = g0
    for _ in range(_K - 1):
        jsel = jnp.argmax(work, axis=1)
        onehot = iota == jsel[:, None]
        work = jnp.where(onehot, -jnp.inf, work)
        g = jax.lax.dot_general(
            onehot.astype(jnp.float32), y1t, (((1,), (0,)), ((), ())),
            preferred_element_type=jnp.float32)
        s1 = s1 + g
        s2 = s2 + g * g
        vmax = jnp.maximum(vmax, g)
        vmin = jnp.minimum(vmin, g)

    amax_ref[0] = y2t + vmax
    amin_ref[0] = y2t + vmin
    kf = float(_K)
    s1p_ref[0, 0, 0] = jnp.sum(s1 + kf * y2t, axis=0)
    s2p_ref[0, 0, 0] = jnp.sum(s2 + 2.0 * y2t * s1 + kf * (y2t * y2t), axis=0)


def _finish_kernel(amax_ref, amin_ref, s1p_ref, s2p_ref, gamma_ref, beta_ref,
                   out_ref, *, total):
    s1 = jnp.sum(s1p_ref[...], axis=(0, 1, 2))
    s2 = jnp.sum(s2p_ref[...], axis=(0, 1, 2))
    mean = s1 / total
    var = s2 / total - mean * mean
    inv = jax.lax.rsqrt(var + 1e-5)
    scale = gamma_ref[0] * inv
    shift = beta_ref[0] - mean * scale
    a = jnp.where(scale[None, :] >= 0.0, amax_ref[0], amin_ref[0])
    v = a * scale[None, :] + shift[None, :]
    v = jnp.where(v > 0.0, v, 0.2 * v)
    out_ref[0] = v.T


def kernel(x, W, gamma, beta):
    B, C, N = x.shape
    O = W.shape[0]
    BR = min(256, N)
    T = N // BR
    wt = jnp.transpose(W)

    amax, amin, s1p, s2p = pl.pallas_call(
        _edge_kernel,
        grid=(B, T),
        in_specs=[
            pl.BlockSpec((1, C, N), lambda b, t: (b, 0, 0)),
            pl.BlockSpec((1, C, BR), lambda b, t: (b, 0, t)),
            pl.BlockSpec((2 * C, O), lambda b, t: (0, 0)),
        ],
        out_specs=[
            pl.BlockSpec((1, BR, O), lambda b, t: (b, t, 0)),
            pl.BlockSpec((1, BR, O), lambda b, t: (b, t, 0)),
            pl.BlockSpec((1, 1, 1, O), lambda b, t: (b, t, 0, 0)),
            pl.BlockSpec((1, 1, 1, O), lambda b, t: (b, t, 0, 0)),
        ],
        out_shape=[
            jax.ShapeDtypeStruct((B, N, O), jnp.float32),
            jax.ShapeDtypeStruct((B, N, O), jnp.float32),
            jax.ShapeDtypeStruct((B, T, 1, O), jnp.float32),
            jax.ShapeDtypeStruct((B, T, 1, O), jnp.float32),
        ],
    )(x, x, wt)

    total = float(B * N * _K)
    out = pl.pallas_call(
        functools.partial(_finish_kernel, total=total),
        grid=(B,),
        in_specs=[
            pl.BlockSpec((1, N, O), lambda b: (b, 0, 0)),
            pl.BlockSpec((1, N, O), lambda b: (b, 0, 0)),
            pl.BlockSpec((B, T, 1, O), lambda b: (0, 0, 0, 0)),
            pl.BlockSpec((B, T, 1, O), lambda b: (0, 0, 0, 0)),
            pl.BlockSpec((1, O), lambda b: (0, 0)),
            pl.BlockSpec((1, O), lambda b: (0, 0)),
        ],
        out_specs=pl.BlockSpec((1, O, N), lambda b: (b, 0, 0)),
        out_shape=jax.ShapeDtypeStruct((B, O, N), jnp.float32),
    )(amax, amin, s1p, s2p, gamma.reshape(1, -1), beta.reshape(1, -1))
    return out


# speedup vs baseline: 10.0658x; 1.0311x over previous
import functools

import jax
import jax.numpy as jnp
from jax.experimental import pallas as pl

_K = 20
_TN = (((0,), (0,)), ((), ()))


def _edge_kernel(x_ref, xb_ref, wt_ref,
                 amax_ref, amin_ref, s1p_ref, s2p_ref):
    xb = x_ref[0]
    xc = xb_ref[0]
    C, N = xb.shape
    BR = xc.shape[1]

    xxf = jnp.sum(xb * xb, axis=0, keepdims=True)
    xxn = jnp.sum(xc * xc, axis=0)[:, None]
    d = 2.0 * jax.lax.dot_general(
        xc, xb, _TN, preferred_element_type=jnp.float32)
    d = d - xxf - xxn

    w1 = wt_ref[0:C, :]
    w2 = wt_ref[C:2 * C, :]
    y1t = jax.lax.dot_general(
        xb, w1, _TN, preferred_element_type=jnp.float32)
    y2t = jax.lax.dot_general(
        xc, w2 - w1, _TN, preferred_element_type=jnp.float32)

    g0 = jax.lax.dot_general(
        xc, w1, _TN, preferred_element_type=jnp.float32)
    iota = jax.lax.broadcasted_iota(jnp.int32, (BR, N), 1)
    riota = jax.lax.broadcasted_iota(jnp.int32, (BR, 1), 0)
    off = pl.program_id(1) * BR
    work = jnp.where(iota == riota + off, -jnp.inf, d)
    s1 = g0
    s2 = g0 * g0
    vmax = g0
    vmin = g0
    for _ in range(_K - 1):
        jsel = jnp.argmax(work, axis=1)
        onehot = iota == jsel[:, None]
        work = jnp.where(onehot, -jnp.inf, work)
        g = jax.lax.dot_general(
            onehot.astype(jnp.float32), y1t, (((1,), (0,)), ((), ())),
            preferred_element_type=jnp.float32)
        s1 = s1 + g
        s2 = s2 + g * g
        vmax = jnp.maximum(vmax, g)
        vmin = jnp.minimum(vmin, g)

    amax_ref[0] = y2t + vmax
    amin_ref[0] = y2t + vmin
    kf = float(_K)
    s1p_ref[0, 0, 0] = jnp.sum(s1 + kf * y2t, axis=0)
    s2p_ref[0, 0, 0] = jnp.sum(s2 + 2.0 * y2t * s1 + kf * (y2t * y2t), axis=0)


def _finish_kernel(amax_ref, amin_ref, s1p_ref, s2p_ref, gamma_ref, beta_ref,
                   out_ref, *, total):
    s1 = jnp.sum(s1p_ref[...], axis=(0, 1, 2))
    s2 = jnp.sum(s2p_ref[...], axis=(0, 1, 2))
    mean = s1 / total
    var = s2 / total - mean * mean
    inv = jax.lax.rsqrt(var + 1e-5)
    scale = gamma_ref[0] * inv
    shift = beta_ref[0] - mean * scale
    a = jnp.where(scale[None, :] >= 0.0, amax_ref[0], amin_ref[0])
    v = a * scale[None, :] + shift[None, :]
    v = jnp.where(v > 0.0, v, 0.2 * v)
    out_ref[0] = v.T


def kernel(x, W, gamma, beta):
    B, C, N = x.shape
    O = W.shape[0]
    BR = min(512, N)
    T = N // BR
    wt = jnp.transpose(W)

    amax, amin, s1p, s2p = pl.pallas_call(
        _edge_kernel,
        grid=(B, T),
        in_specs=[
            pl.BlockSpec((1, C, N), lambda b, t: (b, 0, 0)),
            pl.BlockSpec((1, C, BR), lambda b, t: (b, 0, t)),
            pl.BlockSpec((2 * C, O), lambda b, t: (0, 0)),
        ],
        out_specs=[
            pl.BlockSpec((1, BR, O), lambda b, t: (b, t, 0)),
            pl.BlockSpec((1, BR, O), lambda b, t: (b, t, 0)),
            pl.BlockSpec((1, 1, 1, O), lambda b, t: (b, t, 0, 0)),
            pl.BlockSpec((1, 1, 1, O), lambda b, t: (b, t, 0, 0)),
        ],
        out_shape=[
            jax.ShapeDtypeStruct((B, N, O), jnp.float32),
            jax.ShapeDtypeStruct((B, N, O), jnp.float32),
            jax.ShapeDtypeStruct((B, T, 1, O), jnp.float32),
            jax.ShapeDtypeStruct((B, T, 1, O), jnp.float32),
        ],
    )(x, x, wt)

    total = float(B * N * _K)
    out = pl.pallas_call(
        functools.partial(_finish_kernel, total=total),
        grid=(B,),
        in_specs=[
            pl.BlockSpec((1, N, O), lambda b: (b, 0, 0)),
            pl.BlockSpec((1, N, O), lambda b: (b, 0, 0)),
            pl.BlockSpec((B, T, 1, O), lambda b: (0, 0, 0, 0)),
            pl.BlockSpec((B, T, 1, O), lambda b: (0, 0, 0, 0)),
            pl.BlockSpec((1, O), lambda b: (0, 0)),
            pl.BlockSpec((1, O), lambda b: (0, 0)),
        ],
        out_specs=pl.BlockSpec((1, O, N), lambda b: (b, 0, 0)),
        out_shape=jax.ShapeDtypeStruct((B, O, N), jnp.float32),
    )(amax, amin, s1p, s2p, gamma.reshape(1, -1), beta.reshape(1, -1))
    return out


# speedup vs baseline: 10.3956x; 1.0328x over previous
import functools

import jax
import jax.numpy as jnp
from jax.experimental import pallas as pl

_K = 20
_TN = (((0,), (0,)), ((), ()))


def _edge_kernel(x_ref, xb_ref, wt_ref,
                 amax_ref, amin_ref, s1p_ref, s2p_ref):
    xb = x_ref[0]
    xc = xb_ref[0]
    C, N = xb.shape
    BR = xc.shape[1]

    xxf = jnp.sum(xb * xb, axis=0, keepdims=True)
    xxn = jnp.sum(xc * xc, axis=0)[:, None]
    d = 2.0 * jax.lax.dot_general(
        xc, xb, _TN, preferred_element_type=jnp.float32)
    d = d - xxf - xxn

    w1 = wt_ref[0:C, :]
    w2 = wt_ref[C:2 * C, :]
    y1t = jax.lax.dot_general(
        xb, w1, _TN, preferred_element_type=jnp.float32)
    y2t = jax.lax.dot_general(
        xc, w2 - w1, _TN, preferred_element_type=jnp.float32)

    g0 = jax.lax.dot_general(
        xc, w1, _TN, preferred_element_type=jnp.float32)
    iota = jax.lax.broadcasted_iota(jnp.int32, (BR, N), 1)
    riota = jax.lax.broadcasted_iota(jnp.int32, (BR, 1), 0)
    off = pl.program_id(1) * BR
    work = jnp.where(iota == riota + off, -jnp.inf, d)
    s1 = g0
    s2 = g0 * g0
    vmax = g0
    vmin = g0
    for _ in range(_K - 1):
        jsel = jnp.argmax(work, axis=1)
        onehot = iota == jsel[:, None]
        work = jnp.where(onehot, -jnp.inf, work)
        g = jax.lax.dot_general(
            onehot.astype(jnp.float32), y1t, (((1,), (0,)), ((), ())),
            preferred_element_type=jnp.float32)
        s1 = s1 + g
        s2 = s2 + g * g
        vmax = jnp.maximum(vmax, g)
        vmin = jnp.minimum(vmin, g)

    amax_ref[0] = y2t + vmax
    amin_ref[0] = y2t + vmin
    kf = float(_K)
    s1p_ref[0, 0, 0] = jnp.sum(s1 + kf * y2t, axis=0)
    s2p_ref[0, 0, 0] = jnp.sum(s2 + 2.0 * y2t * s1 + kf * (y2t * y2t), axis=0)


def _finish_kernel(amax_ref, amin_ref, s1p_ref, s2p_ref, gamma_ref, beta_ref,
                   out_ref, *, total):
    s1 = jnp.sum(s1p_ref[...], axis=(0, 1, 2))
    s2 = jnp.sum(s2p_ref[...], axis=(0, 1, 2))
    mean = s1 / total
    var = s2 / total - mean * mean
    inv = jax.lax.rsqrt(var + 1e-5)
    scale = gamma_ref[0] * inv
    shift = beta_ref[0] - mean * scale
    a = jnp.where(scale[None, :] >= 0.0, amax_ref[0], amin_ref[0])
    v = a * scale[None, :] + shift[None, :]
    v = jnp.where(v > 0.0, v, 0.2 * v)
    out_ref[0] = v.T


def kernel(x, W, gamma, beta):
    B, C, N = x.shape
    O = W.shape[0]
    BR = min(1024, N)
    T = N // BR
    wt = jnp.transpose(W)

    amax, amin, s1p, s2p = pl.pallas_call(
        _edge_kernel,
        grid=(B, T),
        in_specs=[
            pl.BlockSpec((1, C, N), lambda b, t: (b, 0, 0)),
            pl.BlockSpec((1, C, BR), lambda b, t: (b, 0, t)),
            pl.BlockSpec((2 * C, O), lambda b, t: (0, 0)),
        ],
        out_specs=[
            pl.BlockSpec((1, BR, O), lambda b, t: (b, t, 0)),
            pl.BlockSpec((1, BR, O), lambda b, t: (b, t, 0)),
            pl.BlockSpec((1, 1, 1, O), lambda b, t: (b, t, 0, 0)),
            pl.BlockSpec((1, 1, 1, O), lambda b, t: (b, t, 0, 0)),
        ],
        out_shape=[
            jax.ShapeDtypeStruct((B, N, O), jnp.float32),
            jax.ShapeDtypeStruct((B, N, O), jnp.float32),
            jax.ShapeDtypeStruct((B, T, 1, O), jnp.float32),
            jax.ShapeDtypeStruct((B, T, 1, O), jnp.float32),
        ],
    )(x, x, wt)

    total = float(B * N * _K)
    out = pl.pallas_call(
        functools.partial(_finish_kernel, total=total),
        grid=(B,),
        in_specs=[
            pl.BlockSpec((1, N, O), lambda b: (b, 0, 0)),
            pl.BlockSpec((1, N, O), lambda b: (b, 0, 0)),
            pl.BlockSpec((B, T, 1, O), lambda b: (0, 0, 0, 0)),
            pl.BlockSpec((B, T, 1, O), lambda b: (0, 0, 0, 0)),
            pl.BlockSpec((1, O), lambda b: (0, 0)),
            pl.BlockSpec((1, O), lambda b: (0, 0)),
        ],
        out_specs=pl.BlockSpec((1, O, N), lambda b: (b, 0, 0)),
        out_shape=jax.ShapeDtypeStruct((B, O, N), jnp.float32),
    )(amax, amin, s1p, s2p, gamma.reshape(1, -1), beta.reshape(1, -1))
    return out
